# baseline (speedup 1.0000x reference)
import functools

import jax
import jax.numpy as jnp
from jax import lax
from jax.experimental import pallas as pl
from jax.experimental.pallas import tpu as pltpu
from jax.experimental.pallas import tpu_sc as plsc

N = 10000
D = 128
G = 64
N_PAD = 10240
E_TOT = 320000
NC, NS = 2, 16
NW = NC * NS
C = 128
CM = C
CH0 = 97
CH1 = 61
E_PAD = NS * (CH0 + CH1) * CM
CHUNKS_PER_W = E_PAD // (NW * C)
T_E = CHUNKS_PER_W * C
SLICE = N_PAD // NS
NIDX = 4
NRB = 2
E_IDX = E_PAD + (NIDX - 1) * CM

@functools.cache
def _mesh():
    return plsc.VectorSubcoreMesh(core_axis_name="c", subcore_axis_name="s",
                                  num_cores=NC, num_subcores=NS)


def _fill(buf, rows, value):
    def fbody(j, carry):
        for f in range(D // 16):
            buf[j, pl.ds(16 * f, 16)] = jnp.full((16,), value, jnp.float32)
        return carry

    lax.fori_loop(0, rows, fbody, 0)


def _zero_acc(buf, acc_sh, s):
    _fill(buf, CM, 0.0)
    for t in range(SLICE // CM):
        pltpu.sync_copy(buf, acc_sh.at[pl.ds(s * SLICE + t * CM, CM)])


def _deg_body(dst_hbm, out_hbm, dst_v, ones_v, acc_sh):
    c = lax.axis_index("c")
    s = lax.axis_index("s")
    wid = s * NC + c
    _zero_acc(ones_v, acc_sh, s)
    _fill(ones_v, C, 1.0)
    plsc.subcore_barrier()

    def body(i, carry):
        base = wid * T_E + i * C
        pltpu.sync_copy(dst_hbm.at[pl.ds(base, C)], dst_v)
        pltpu.sync_copy(ones_v, acc_sh.at[dst_v], add=True)
        return carry

    lax.fori_loop(0, CHUNKS_PER_W, body, 0)
    plsc.subcore_barrier()
    pltpu.sync_copy(acc_sh.at[pl.ds(s * SLICE, SLICE)],
                    out_hbm.at[pl.ds(c * N_PAD + s * SLICE, SLICE)])


@functools.cache
def _deg_kernel():
    return pl.kernel(
        _deg_body,
        out_type=jax.ShapeDtypeStruct((NC * N_PAD, D), jnp.float32),
        mesh=_mesh(),
        scratch_types=[
            pltpu.VMEM((C,), jnp.int32),
            pltpu.VMEM((C, D), jnp.float32),
            pltpu.VMEM_SHARED((N_PAD, D), jnp.float32),
        ],
    )


def _msg_body(g_hbm, src_hbm, dst_hbm, out0_hbm, out1_hbm,
              src_b, dst_b, isems, rbufs, gsems, ssems, acc_sh):
    c = lax.axis_index("c")
    s = lax.axis_index("s")


    def pipeline(base, ch):
        def start_idx(i, q):
            pltpu.async_copy(src_hbm.at[pl.ds(base + i * CM, CM)], src_b[q],
                             isems[q])
            pltpu.async_copy(dst_hbm.at[pl.ds(base + i * CM, CM)], dst_b[q],
                             isems[q])

        def wait_idx(q):
            pltpu.make_async_copy(src_hbm.at[pl.ds(0, CM)], src_b[q],
                                  isems[q]).wait()
            pltpu.make_async_copy(dst_hbm.at[pl.ds(0, CM)], dst_b[q],
                                  isems[q]).wait()

        def start_gather(q, b):
            pltpu.async_copy(g_hbm.at[src_b[q]], rbufs[b], gsems[b])

        def wait_gather(b):
            pltpu.make_async_copy(g_hbm.at[pl.ds(0, CM)], rbufs[b],
                                  gsems[b]).wait()

        def start_scatter(q, b):
            pltpu.async_copy(rbufs[b], acc_sh.at[dst_b[q]], ssems[b],
                             add=True)

        def wait_scatter(b):
            pltpu.make_async_copy(rbufs[b], acc_sh.at[pl.ds(0, CM)],
                                  ssems[b]).wait()

        for q in range(NIDX):
            start_idx(q, q)
        for b in range(NRB):
            wait_idx(b)
            start_gather(b, b)

        def body(k, carry):
            i0 = k * NIDX
            for b in range(NIDX):
                i = i0 + b
                rb = b % NRB
                wait_gather(rb)
                start_scatter(b, rb)
                wait_idx((b + NRB) % NIDX)
                wait_scatter(rb)
                start_idx(i + NIDX, b)
                start_gather((b + NRB) % NIDX, rb)
            return carry

        n_full = (ch - 1) // NIDX
        lax.fori_loop(0, n_full, body, 0)

        last = n_full * NIDX
        wait_gather(last % NRB)
        start_scatter(last % NIDX, last % NRB)
        wait_gather((last + 1) % NRB)
        wait_scatter(last % NRB)
        for i in range(last + 2, last + NIDX):
            wait_idx(i % NIDX)

    _zero_acc(rbufs[0], acc_sh, s)
    plsc.subcore_barrier()

    @pl.when(c == 0)
    def _core0():
        pipeline(s * CH0 * CM, CH0)

    @pl.when(c == 1)
    def _core1():
        pipeline((NS * CH0 + s * CH1) * CM, CH1)

    plsc.subcore_barrier()

    @pl.when(c == 0)
    def _dump0():
        pltpu.sync_copy(acc_sh.at[pl.ds(s * SLICE, SLICE)],
                        out0_hbm.at[pl.ds(s * SLICE, SLICE)])

    @pl.when(c == 1)
    def _dump1():
        pltpu.sync_copy(acc_sh.at[pl.ds(s * SLICE, SLICE)],
                        out1_hbm.at[pl.ds(s * SLICE, SLICE)])


@functools.cache
def _msg_kernel():
    return pl.kernel(
        _msg_body,
        out_type=(jax.ShapeDtypeStruct((N_PAD, D), jnp.float32),
                  jax.ShapeDtypeStruct((N_PAD, D), jnp.float32)),
        mesh=_mesh(),
        scratch_types=[
            [pltpu.VMEM((CM,), jnp.int32)] * NIDX,
            [pltpu.VMEM((CM,), jnp.int32)] * NIDX,
            [pltpu.SemaphoreType.DMA] * NIDX,
            [pltpu.VMEM((CM, D), jnp.float32)] * NRB,
            [pltpu.SemaphoreType.DMA] * NRB,
            [pltpu.SemaphoreType.DMA] * NRB,
            pltpu.VMEM_SHARED((N_PAD, D), jnp.float32),
        ],
    )


_BLK = 1000
_GRID = N // _BLK


def _dense_first_body(x_ref, dinv_ref, wt_ref, g_ref):
    g_ref[...] = jnp.dot(x_ref[...], wt_ref[...],
                         preferred_element_type=jnp.float32) * dinv_ref[...]


def _dense_mid_body(a0_ref, a1_ref, g_ref, dinv_ref, b_ref, wt_ref, o_ref):
    t = (a0_ref[...] + a1_ref[...] + g_ref[...]) * dinv_ref[...] + b_ref[...]
    t = jnp.maximum(t, 0.0)
    o_ref[...] = jnp.dot(t, wt_ref[...],
                         preferred_element_type=jnp.float32) * dinv_ref[...]


def _pool_body(a0_ref, a1_ref, g_ref, dinv_ref, b_ref, batch_ref, out_ref,
               sums_s, counts_s):
    i = pl.program_id(0)
    h = (a0_ref[...] + a1_ref[...] + g_ref[...]) * dinv_ref[...] + b_ref[...]
    gids = lax.broadcasted_iota(jnp.int32, (G, _BLK), 0)
    onehot = (batch_ref[...].reshape(1, _BLK) == gids).astype(jnp.float32)
    part = jnp.dot(onehot, h, preferred_element_type=jnp.float32)
    cnt = jnp.broadcast_to(jnp.sum(onehot, axis=1, keepdims=True), (G, D))

    @pl.when(i == 0)
    def _init():
        sums_s[...] = part
        counts_s[...] = cnt

    @pl.when(i > 0)
    def _acc():
        sums_s[...] += part
        counts_s[...] += cnt

    @pl.when(i == _GRID - 1)
    def _fin():
        out_ref[...] = sums_s[...] / jnp.maximum(counts_s[...], 1.0)


_row_spec = pl.BlockSpec((_BLK, D), lambda i: (i, 0))
_col_spec = pl.BlockSpec((_BLK, 1), lambda i: (i, 0))
_full_spec = pl.BlockSpec((D, D), lambda i: (0, 0))
_bias_spec = pl.BlockSpec((1, D), lambda i: (0, 0))

_dense_first = pl.pallas_call(
    _dense_first_body,
    grid=(_GRID,),
    in_specs=[_row_spec, _col_spec, _full_spec],
    out_specs=_row_spec,
    out_shape=jax.ShapeDtypeStruct((N_PAD, D), jnp.float32),
)

_dense_mid = pl.pallas_call(
    _dense_mid_body,
    grid=(_GRID,),
    in_specs=[_row_spec, _row_spec, _row_spec, _col_spec, _bias_spec,
              _full_spec],
    out_specs=_row_spec,
    out_shape=jax.ShapeDtypeStruct((N_PAD, D), jnp.float32),
)

_pool = pl.pallas_call(
    _pool_body,
    grid=(_GRID,),
    in_specs=[_row_spec, _row_spec, _row_spec, _col_spec, _bias_spec,
              pl.BlockSpec((_BLK, 1), lambda i: (i, 0))],
    out_specs=pl.BlockSpec((G, D), lambda i: (0, 0)),
    out_shape=jax.ShapeDtypeStruct((G, D), jnp.float32),
    scratch_shapes=[pltpu.VMEM((G, D), jnp.float32),
                    pltpu.VMEM((G, D), jnp.float32)],
)


def _make_edges(edge_index):
    pad = jnp.full((E_IDX - E_TOT,), N, jnp.int32)
    src = jnp.concatenate([edge_index[0].astype(jnp.int32), pad])
    dst = jnp.concatenate([edge_index[1].astype(jnp.int32), pad])
    return src, dst


@jax.jit
def _run_deg(edge_index):
    _, dst = _make_edges(edge_index)
    deg_parts = _deg_kernel()(dst)
    return deg_parts[:N, 0] + deg_parts[N_PAD:N_PAD + N, 0]


@jax.jit
def _run_prep(deg, x, edge_index, W1):
    src, dst = _make_edges(edge_index)
    dinv = lax.rsqrt(deg + 1.0).reshape(N, 1)
    g = _dense_first(x, dinv, W1.T)
    return g, src, dst, dinv


@jax.jit
def _run_msg1(g, src, dst, dinv, b, Wn):
    a0, a1 = _msg_kernel()(g, src, dst)
    return _dense_mid(a0, a1, g, dinv, b.reshape(1, D), Wn.T)


_run_msg2 = _run_msg1


@jax.jit
def _run_msg3(g, src, dst, dinv, b, batch):
    a0, a1 = _msg_kernel()(g, src, dst)
    return _pool(a0, a1, g, dinv, b.reshape(1, D),
                 batch.astype(jnp.int32).reshape(N, 1))


def kernel(x, edge_index, batch, W1, b1, W2, b2, W3, b3):
    deg = _run_deg(edge_index)
    g, src, dst, dinv = _run_prep(deg, x, edge_index, W1)
    g = _run_msg1(g, src, dst, dinv, b1, W2)
    g = _run_msg2(g, src, dst, dinv, b2, W3)
    return _run_msg3(g, src, dst, dinv, b3, batch)


# speedup vs baseline: 1.0251x; 1.0251x over previous
import functools

import jax
import jax.numpy as jnp
from jax import lax
from jax.experimental import pallas as pl
from jax.experimental.pallas import tpu as pltpu
from jax.experimental.pallas import tpu_sc as plsc

N = 10000
D = 128
G = 64
N_PAD = 10240
E_TOT = 320000
NC, NS = 2, 16
NW = NC * NS
C = 128
CM = C
CH0 = 97
CH1 = 61
E_PAD = NS * (CH0 + CH1) * CM
CHUNKS_PER_W = E_PAD // (NW * C)
T_E = CHUNKS_PER_W * C
SLICE = N_PAD // NS
NIDX = 4
NRB = 2
E_IDX = E_PAD + (NIDX - 1) * CM

@functools.cache
def _mesh():
    return plsc.VectorSubcoreMesh(core_axis_name="c", subcore_axis_name="s",
                                  num_cores=NC, num_subcores=NS)


def _fill(buf, rows, value):
    def fbody(j, carry):
        for f in range(D // 16):
            buf[j, pl.ds(16 * f, 16)] = jnp.full((16,), value, jnp.float32)
        return carry

    lax.fori_loop(0, rows, fbody, 0)


def _zero_acc(buf, acc_sh, s):
    _fill(buf, CM, 0.0)
    for t in range(SLICE // CM):
        pltpu.sync_copy(buf, acc_sh.at[pl.ds(s * SLICE + t * CM, CM)])


def _deg_body(dst_hbm, out_hbm, dst_v, ones_v, acc_sh):
    c = lax.axis_index("c")
    s = lax.axis_index("s")
    wid = s * NC + c
    _zero_acc(ones_v, acc_sh, s)
    _fill(ones_v, C, 1.0)
    plsc.subcore_barrier()

    def body(i, carry):
        base = wid * T_E + i * C
        pltpu.sync_copy(dst_hbm.at[pl.ds(base, C)], dst_v)
        pltpu.sync_copy(ones_v, acc_sh.at[dst_v], add=True)
        return carry

    lax.fori_loop(0, CHUNKS_PER_W, body, 0)
    plsc.subcore_barrier()
    pltpu.sync_copy(acc_sh.at[pl.ds(s * SLICE, SLICE)],
                    out_hbm.at[pl.ds(c * N_PAD + s * SLICE, SLICE)])


@functools.cache
def _deg_kernel():
    return pl.kernel(
        _deg_body,
        out_type=jax.ShapeDtypeStruct((NC * N_PAD, D), jnp.float32),
        mesh=_mesh(),
        scratch_types=[
            pltpu.VMEM((C,), jnp.int32),
            pltpu.VMEM((C, D), jnp.float32),
            pltpu.VMEM_SHARED((N_PAD, D), jnp.float32),
        ],
    )


def _msg_body(g_hbm, src_hbm, dst_hbm, out0_hbm, out1_hbm,
              src_b, dst_b, isems, rbufs, gsems, ssems, acc_sh):
    c = lax.axis_index("c")
    s = lax.axis_index("s")


    def pipeline(base, ch):
        def start_idx(i, q):
            pltpu.async_copy(src_hbm.at[pl.ds(base + i * CM, CM)], src_b[q],
                             isems[q])
            pltpu.async_copy(dst_hbm.at[pl.ds(base + i * CM, CM)], dst_b[q],
                             isems[q])

        def wait_idx(q):
            pltpu.make_async_copy(src_hbm.at[pl.ds(0, CM)], src_b[q],
                                  isems[q]).wait()
            pltpu.make_async_copy(dst_hbm.at[pl.ds(0, CM)], dst_b[q],
                                  isems[q]).wait()

        def start_gather(q, b):
            pltpu.async_copy(g_hbm.at[src_b[q]], rbufs[b], gsems[b])

        def wait_gather(b):
            pltpu.make_async_copy(g_hbm.at[pl.ds(0, CM)], rbufs[b],
                                  gsems[b]).wait()

        def start_scatter(q, b):
            pltpu.async_copy(rbufs[b], acc_sh.at[dst_b[q]], ssems[b],
                             add=True)

        def wait_scatter(b):
            pltpu.make_async_copy(rbufs[b], acc_sh.at[pl.ds(0, CM)],
                                  ssems[b]).wait()

        for q in range(NIDX):
            start_idx(q, q)
        for b in range(NRB):
            wait_idx(b)
            start_gather(b, b)

        def body(k, carry):
            i0 = k * NIDX
            for b in range(NIDX):
                i = i0 + b
                rb = b % NRB
                wait_gather(rb)
                start_scatter(b, rb)
                wait_idx((b + NRB) % NIDX)
                wait_scatter(rb)
                start_idx(i + NIDX, b)
                start_gather((b + NRB) % NIDX, rb)
            return carry

        n_full = (ch - 1) // NIDX
        lax.fori_loop(0, n_full, body, 0)

        last = n_full * NIDX
        wait_gather(last % NRB)
        start_scatter(last % NIDX, last % NRB)
        wait_gather((last + 1) % NRB)
        wait_scatter(last % NRB)
        for i in range(last + 2, last + NIDX):
            wait_idx(i % NIDX)

    _zero_acc(rbufs[0], acc_sh, s)
    plsc.subcore_barrier()

    @pl.when(c == 0)
    def _core0():
        pipeline(s * CH0 * CM, CH0)

    @pl.when(c == 1)
    def _core1():
        pipeline((NS * CH0 + s * CH1) * CM, CH1)

    plsc.subcore_barrier()

    @pl.when(c == 0)
    def _dump0():
        pltpu.sync_copy(acc_sh.at[pl.ds(s * SLICE, SLICE)],
                        out0_hbm.at[pl.ds(s * SLICE, SLICE)])

    @pl.when(c == 1)
    def _dump1():
        pltpu.sync_copy(acc_sh.at[pl.ds(s * SLICE, SLICE)],
                        out1_hbm.at[pl.ds(s * SLICE, SLICE)])


@functools.cache
def _msg_kernel():
    return pl.kernel(
        _msg_body,
        out_type=(jax.ShapeDtypeStruct((N_PAD, D), jnp.float32),
                  jax.ShapeDtypeStruct((N_PAD, D), jnp.float32)),
        mesh=_mesh(),
        scratch_types=[
            [pltpu.VMEM((CM,), jnp.int32)] * NIDX,
            [pltpu.VMEM((CM,), jnp.int32)] * NIDX,
            [pltpu.SemaphoreType.DMA] * NIDX,
            [pltpu.VMEM((CM, D), jnp.float32)] * NRB,
            [pltpu.SemaphoreType.DMA] * NRB,
            [pltpu.SemaphoreType.DMA] * NRB,
            pltpu.VMEM_SHARED((N_PAD, D), jnp.float32),
        ],
    )


_BLK = 1024
_GRID = N_PAD // _BLK


def _dense_first_body(x_ref, dinv_ref, wt_ref, g_ref):
    g_ref[...] = jnp.dot(x_ref[...], wt_ref[...],
                         preferred_element_type=jnp.float32) * dinv_ref[...]


def _dense_mid_body(a0_ref, a1_ref, g_ref, dinv_ref, b_ref, wt_ref, o_ref):
    t = (a0_ref[...] + a1_ref[...] + g_ref[...]) * dinv_ref[...] + b_ref[...]
    t = jnp.maximum(t, 0.0)
    o_ref[...] = jnp.dot(t, wt_ref[...],
                         preferred_element_type=jnp.float32) * dinv_ref[...]


def _pool_body(a0_ref, a1_ref, g_ref, dinv_ref, b_ref, batch_ref, out_ref,
               sums_s, counts_s):
    i = pl.program_id(0)
    h = (a0_ref[...] + a1_ref[...] + g_ref[...]) * dinv_ref[...] + b_ref[...]
    gids = lax.broadcasted_iota(jnp.int32, (G, _BLK), 0)
    onehot = (batch_ref[...].reshape(1, _BLK) == gids).astype(jnp.float32)
    part = jnp.dot(onehot, h, preferred_element_type=jnp.float32)
    cnt = jnp.broadcast_to(jnp.sum(onehot, axis=1, keepdims=True), (G, D))

    @pl.when(i == 0)
    def _init():
        sums_s[...] = part
        counts_s[...] = cnt

    @pl.when(i > 0)
    def _acc():
        sums_s[...] += part
        counts_s[...] += cnt

    @pl.when(i == _GRID - 1)
    def _fin():
        out_ref[...] = sums_s[...] / jnp.maximum(counts_s[...], 1.0)


_row_spec = pl.BlockSpec((_BLK, D), lambda i: (i, 0))
_col_spec = pl.BlockSpec((_BLK, 1), lambda i: (i, 0))
_full_spec = pl.BlockSpec((D, D), lambda i: (0, 0))
_bias_spec = pl.BlockSpec((1, D), lambda i: (0, 0))

_dense_first = pl.pallas_call(
    _dense_first_body,
    grid=(_GRID,),
    in_specs=[_row_spec, _col_spec, _full_spec],
    out_specs=_row_spec,
    out_shape=jax.ShapeDtypeStruct((N_PAD, D), jnp.float32),
)

_dense_mid = pl.pallas_call(
    _dense_mid_body,
    grid=(_GRID,),
    in_specs=[_row_spec, _row_spec, _row_spec, _col_spec, _bias_spec,
              _full_spec],
    out_specs=_row_spec,
    out_shape=jax.ShapeDtypeStruct((N_PAD, D), jnp.float32),
)

_pool = pl.pallas_call(
    _pool_body,
    grid=(_GRID,),
    in_specs=[_row_spec, _row_spec, _row_spec, _col_spec, _bias_spec,
              pl.BlockSpec((_BLK, 1), lambda i: (i, 0))],
    out_specs=pl.BlockSpec((G, D), lambda i: (0, 0)),
    out_shape=jax.ShapeDtypeStruct((G, D), jnp.float32),
    scratch_shapes=[pltpu.VMEM((G, D), jnp.float32),
                    pltpu.VMEM((G, D), jnp.float32)],
)


def _make_edges(edge_index):
    pad = jnp.full((E_IDX - E_TOT,), N, jnp.int32)
    src = jnp.concatenate([edge_index[0].astype(jnp.int32), pad])
    dst = jnp.concatenate([edge_index[1].astype(jnp.int32), pad])
    return src, dst


@jax.jit
def _run_deg(edge_index):
    _, dst = _make_edges(edge_index)
    deg_parts = _deg_kernel()(dst)
    return deg_parts[:N_PAD, 0] + deg_parts[N_PAD:, 0]


@jax.jit
def _run_prep(deg, x, edge_index, W1):
    src, dst = _make_edges(edge_index)
    x_p = jnp.zeros((N_PAD, D), jnp.float32).at[:N].set(x)
    dinv = lax.rsqrt(deg + 1.0).reshape(N_PAD, 1)
    g = _dense_first(x_p, dinv, W1.T)
    return g, src, dst, dinv


@jax.jit
def _run_msg1(g, src, dst, dinv, b, Wn):
    a0, a1 = _msg_kernel()(g, src, dst)
    return _dense_mid(a0, a1, g, dinv, b.reshape(1, D), Wn.T)


_run_msg2 = _run_msg1


@jax.jit
def _run_msg3(g, src, dst, dinv, b, batch):
    a0, a1 = _msg_kernel()(g, src, dst)
    batch_p = jnp.full((N_PAD, 1), G, jnp.int32).at[:N, 0].set(
        batch.astype(jnp.int32))
    return _pool(a0, a1, g, dinv, b.reshape(1, D), batch_p)


def kernel(x, edge_index, batch, W1, b1, W2, b2, W3, b3):
    deg = _run_deg(edge_index)
    g, src, dst, dinv = _run_prep(deg, x, edge_index, W1)
    g = _run_msg1(g, src, dst, dinv, b1, W2)
    g = _run_msg2(g, src, dst, dinv, b2, W3)
    return _run_msg3(g, src, dst, dinv, b3, batch)


# speedup vs baseline: 1.6170x; 1.5775x over previous
import functools

import jax
import jax.numpy as jnp
from jax import lax
from jax.experimental import pallas as pl
from jax.experimental.pallas import tpu as pltpu
from jax.experimental.pallas import tpu_sc as plsc

N = 10000
D = 128
G = 64
N_PAD = 10240
E_TOT = 320000
NC, NS = 2, 16
NW = NC * NS
C = 128
CM = C
CH0 = 97
CH1 = 61
E_PAD = NS * (CH0 + CH1) * CM
CHUNKS_PER_W = E_PAD // (NW * C)
T_E = CHUNKS_PER_W * C
SLICE = N_PAD // NS
NIDX = 4
NRB = 2
E_IDX = E_PAD + (NIDX - 1) * CM

@functools.cache
def _mesh():
    return plsc.VectorSubcoreMesh(core_axis_name="c", subcore_axis_name="s",
                                  num_cores=NC, num_subcores=NS)


def _fill(buf, rows, value):
    def fbody(j, carry):
        for f in range(D // 16):
            buf[j, pl.ds(16 * f, 16)] = jnp.full((16,), value, jnp.float32)
        return carry

    lax.fori_loop(0, rows, fbody, 0)


def _zero_acc(buf, acc_sh, s):
    _fill(buf, CM, 0.0)
    for t in range(SLICE // CM):
        pltpu.sync_copy(buf, acc_sh.at[pl.ds(s * SLICE + t * CM, CM)])


def _deg_body(dst_hbm, out_hbm, dst_v, ones_v, acc_sh):
    c = lax.axis_index("c")
    s = lax.axis_index("s")
    wid = s * NC + c
    _zero_acc(ones_v, acc_sh, s)
    _fill(ones_v, C, 1.0)
    plsc.subcore_barrier()

    def body(i, carry):
        base = wid * T_E + i * C
        pltpu.sync_copy(dst_hbm.at[pl.ds(base, C)], dst_v)
        pltpu.sync_copy(ones_v, acc_sh.at[dst_v], add=True)
        return carry

    lax.fori_loop(0, CHUNKS_PER_W, body, 0)
    plsc.subcore_barrier()
    pltpu.sync_copy(acc_sh.at[pl.ds(s * SLICE, SLICE)],
                    out_hbm.at[pl.ds(c * N_PAD + s * SLICE, SLICE)])


@functools.cache
def _deg_kernel():
    return pl.kernel(
        _deg_body,
        out_type=jax.ShapeDtypeStruct((NC * N_PAD, D), jnp.float32),
        mesh=_mesh(),
        scratch_types=[
            pltpu.VMEM((C,), jnp.int32),
            pltpu.VMEM((C, D), jnp.float32),
            pltpu.VMEM_SHARED((N_PAD, D), jnp.float32),
        ],
    )


def _msg_body(g_hbm, src_hbm, dst_hbm, out0_hbm, out1_hbm,
              src_b, dst_b, isems, rbufs, gsems, ssems, acc_sh):
    c = lax.axis_index("c")
    s = lax.axis_index("s")


    def pipeline(base, ch):
        def start_idx(i, q):
            pltpu.async_copy(src_hbm.at[pl.ds(base + i * CM, CM)], src_b[q],
                             isems[q])
            pltpu.async_copy(dst_hbm.at[pl.ds(base + i * CM, CM)], dst_b[q],
                             isems[q])

        def wait_idx(q):
            pltpu.make_async_copy(src_hbm.at[pl.ds(0, CM)], src_b[q],
                                  isems[q]).wait()
            pltpu.make_async_copy(dst_hbm.at[pl.ds(0, CM)], dst_b[q],
                                  isems[q]).wait()

        def start_gather(q, b):
            pltpu.async_copy(g_hbm.at[src_b[q]], rbufs[b], gsems[b])

        def wait_gather(b):
            pltpu.make_async_copy(g_hbm.at[pl.ds(0, CM)], rbufs[b],
                                  gsems[b]).wait()

        def start_scatter(q, b):
            pltpu.async_copy(rbufs[b], acc_sh.at[dst_b[q]], ssems[b],
                             add=True)

        def wait_scatter(b):
            pltpu.make_async_copy(rbufs[b], acc_sh.at[pl.ds(0, CM)],
                                  ssems[b]).wait()

        for q in range(NIDX):
            start_idx(q, q)
        for b in range(NRB):
            wait_idx(b)
            start_gather(b, b)

        def body(k, carry):
            i0 = k * NIDX
            for b in range(NIDX):
                i = i0 + b
                rb = b % NRB
                wait_gather(rb)
                start_scatter(b, rb)
                wait_idx((b + NRB) % NIDX)
                wait_scatter(rb)
                start_idx(i + NIDX, b)
                start_gather((b + NRB) % NIDX, rb)
            return carry

        n_full = (ch - 1) // NIDX
        lax.fori_loop(0, n_full, body, 0)

        last = n_full * NIDX
        wait_gather(last % NRB)
        start_scatter(last % NIDX, last % NRB)
        wait_gather((last + 1) % NRB)
        wait_scatter(last % NRB)
        for i in range(last + 2, last + NIDX):
            wait_idx(i % NIDX)

    _zero_acc(rbufs[0], acc_sh, s)
    plsc.subcore_barrier()

    @pl.when(c == 0)
    def _core0():
        pipeline(s * CH0 * CM, CH0)

    @pl.when(c == 1)
    def _core1():
        pipeline((NS * CH0 + s * CH1) * CM, CH1)

    plsc.subcore_barrier()

    @pl.when(c == 0)
    def _dump0():
        pltpu.sync_copy(acc_sh.at[pl.ds(s * SLICE, SLICE)],
                        out0_hbm.at[pl.ds(s * SLICE, SLICE)])

    @pl.when(c == 1)
    def _dump1():
        pltpu.sync_copy(acc_sh.at[pl.ds(s * SLICE, SLICE)],
                        out1_hbm.at[pl.ds(s * SLICE, SLICE)])


@functools.cache
def _msg_kernel():
    return pl.kernel(
        _msg_body,
        out_type=(jax.ShapeDtypeStruct((N_PAD, D), jnp.float32),
                  jax.ShapeDtypeStruct((N_PAD, D), jnp.float32)),
        mesh=_mesh(),
        scratch_types=[
            [pltpu.VMEM((CM,), jnp.int32)] * NIDX,
            [pltpu.VMEM((CM,), jnp.int32)] * NIDX,
            [pltpu.SemaphoreType.DMA] * NIDX,
            [pltpu.VMEM((CM, D), jnp.float32)] * NRB,
            [pltpu.SemaphoreType.DMA] * NRB,
            [pltpu.SemaphoreType.DMA] * NRB,
            pltpu.VMEM_SHARED((N_PAD, D), jnp.float32),
        ],
    )


_BLK = 1024
_GRID = N_PAD // _BLK


def _dense_first_body(x_ref, dinv_ref, wt_ref, g_ref):
    g_ref[...] = jnp.dot(x_ref[...], wt_ref[...],
                         preferred_element_type=jnp.float32) * dinv_ref[...]


def _dense_mid_body(a0_ref, a1_ref, g_ref, dinv_ref, b_ref, wt_ref, o_ref):
    t = (a0_ref[...] + a1_ref[...] + g_ref[...]) * dinv_ref[...] + b_ref[...]
    t = jnp.maximum(t, 0.0)
    o_ref[...] = jnp.dot(t, wt_ref[...],
                         preferred_element_type=jnp.float32) * dinv_ref[...]


def _pool_body(a0_ref, a1_ref, g_ref, dinv_ref, b_ref, batch_ref, out_ref,
               sums_s, counts_s):
    i = pl.program_id(0)
    h = (a0_ref[...] + a1_ref[...] + g_ref[...]) * dinv_ref[...] + b_ref[...]
    gids = lax.broadcasted_iota(jnp.int32, (G, _BLK), 0)
    onehot = (batch_ref[...].reshape(1, _BLK) == gids).astype(jnp.float32)
    part = jnp.dot(onehot, h, preferred_element_type=jnp.float32)
    cnt = jnp.broadcast_to(jnp.sum(onehot, axis=1, keepdims=True), (G, D))

    @pl.when(i == 0)
    def _init():
        sums_s[...] = part
        counts_s[...] = cnt

    @pl.when(i > 0)
    def _acc():
        sums_s[...] += part
        counts_s[...] += cnt

    @pl.when(i == _GRID - 1)
    def _fin():
        out_ref[...] = sums_s[...] / jnp.maximum(counts_s[...], 1.0)


_row_spec = pl.BlockSpec((_BLK, D), lambda i: (i, 0))
_col_spec = pl.BlockSpec((_BLK, 1), lambda i: (i, 0))
_full_spec = pl.BlockSpec((D, D), lambda i: (0, 0))
_bias_spec = pl.BlockSpec((1, D), lambda i: (0, 0))

_dense_first = pl.pallas_call(
    _dense_first_body,
    grid=(_GRID,),
    in_specs=[_row_spec, _col_spec, _full_spec],
    out_specs=_row_spec,
    out_shape=jax.ShapeDtypeStruct((N_PAD, D), jnp.float32),
)

_dense_mid = pl.pallas_call(
    _dense_mid_body,
    grid=(_GRID,),
    in_specs=[_row_spec, _row_spec, _row_spec, _col_spec, _bias_spec,
              _full_spec],
    out_specs=_row_spec,
    out_shape=jax.ShapeDtypeStruct((N_PAD, D), jnp.float32),
)

_pool = pl.pallas_call(
    _pool_body,
    grid=(_GRID,),
    in_specs=[_row_spec, _row_spec, _row_spec, _col_spec, _bias_spec,
              pl.BlockSpec((_BLK, 1), lambda i: (i, 0))],
    out_specs=pl.BlockSpec((G, D), lambda i: (0, 0)),
    out_shape=jax.ShapeDtypeStruct((G, D), jnp.float32),
    scratch_shapes=[pltpu.VMEM((G, D), jnp.float32),
                    pltpu.VMEM((G, D), jnp.float32)],
)


def _make_edges(edge_index):
    pad = N + (jnp.arange(E_IDX - E_TOT, dtype=jnp.int32) % (N_PAD - N))
    src = jnp.concatenate([edge_index[0].astype(jnp.int32), pad])
    dst = jnp.concatenate([edge_index[1].astype(jnp.int32), pad])
    return src, dst


@jax.jit
def _run_deg(edge_index):
    _, dst = _make_edges(edge_index)
    deg_parts = _deg_kernel()(dst)
    return deg_parts[:N_PAD, 0] + deg_parts[N_PAD:, 0]


@jax.jit
def _run_prep(deg, x, edge_index, W1):
    src, dst = _make_edges(edge_index)
    x_p = jnp.zeros((N_PAD, D), jnp.float32).at[:N].set(x)
    dinv = lax.rsqrt(deg + 1.0).reshape(N_PAD, 1)
    g = _dense_first(x_p, dinv, W1.T)
    return g, src, dst, dinv


@jax.jit
def _run_msg1(g, src, dst, dinv, b, Wn):
    a0, a1 = _msg_kernel()(g, src, dst)
    return _dense_mid(a0, a1, g, dinv, b.reshape(1, D), Wn.T)


_run_msg2 = _run_msg1


@jax.jit
def _run_msg3(g, src, dst, dinv, b, batch):
    a0, a1 = _msg_kernel()(g, src, dst)
    batch_p = jnp.full((N_PAD, 1), G, jnp.int32).at[:N, 0].set(
        batch.astype(jnp.int32))
    return _pool(a0, a1, g, dinv, b.reshape(1, D), batch_p)


def kernel(x, edge_index, batch, W1, b1, W2, b2, W3, b3):
    deg = _run_deg(edge_index)
    g, src, dst, dinv = _run_prep(deg, x, edge_index, W1)
    g = _run_msg1(g, src, dst, dinv, b1, W2)
    g = _run_msg2(g, src, dst, dinv, b2, W3)
    return _run_msg3(g, src, dst, dinv, b3, batch)


# speedup vs baseline: 1.7719x; 1.0958x over previous
import functools

import jax
import jax.numpy as jnp
from jax import lax
from jax.experimental import pallas as pl
from jax.experimental.pallas import tpu as pltpu
from jax.experimental.pallas import tpu_sc as plsc

N = 10000
D = 128
G = 64
N_PAD = 10240
E_TOT = 320000
NC, NS = 2, 16
NW = NC * NS
C = 128
CM = C
CH0 = 81
CH1 = 77
E_PAD = NS * (CH0 + CH1) * CM
CHUNKS_PER_W = E_PAD // (NW * C)
T_E = CHUNKS_PER_W * C
SLICE = N_PAD // NS
NIDX = 4
NRB = 2
E_IDX = E_PAD + (NIDX - 1) * CM

@functools.cache
def _mesh():
    return plsc.VectorSubcoreMesh(core_axis_name="c", subcore_axis_name="s",
                                  num_cores=NC, num_subcores=NS)


def _fill(buf, rows, value):
    def fbody(j, carry):
        for f in range(D // 16):
            buf[j, pl.ds(16 * f, 16)] = jnp.full((16,), value, jnp.float32)
        return carry

    lax.fori_loop(0, rows, fbody, 0)


def _zero_acc(buf, acc_sh, s):
    _fill(buf, CM, 0.0)
    for t in range(SLICE // CM):
        pltpu.sync_copy(buf, acc_sh.at[pl.ds(s * SLICE + t * CM, CM)])


def _deg_body(dst_hbm, out_hbm, dst_v, ones_v, acc_sh):
    c = lax.axis_index("c")
    s = lax.axis_index("s")
    wid = s * NC + c
    _zero_acc(ones_v, acc_sh, s)
    _fill(ones_v, C, 1.0)
    plsc.subcore_barrier()

    def body(i, carry):
        base = wid * T_E + i * C
        pltpu.sync_copy(dst_hbm.at[pl.ds(base, C)], dst_v)
        pltpu.sync_copy(ones_v, acc_sh.at[dst_v], add=True)
        return carry

    lax.fori_loop(0, CHUNKS_PER_W, body, 0)
    plsc.subcore_barrier()
    pltpu.sync_copy(acc_sh.at[pl.ds(s * SLICE, SLICE)],
                    out_hbm.at[pl.ds(c * N_PAD + s * SLICE, SLICE)])


@functools.cache
def _deg_kernel():
    return pl.kernel(
        _deg_body,
        out_type=jax.ShapeDtypeStruct((NC * N_PAD, D), jnp.float32),
        mesh=_mesh(),
        scratch_types=[
            pltpu.VMEM((C,), jnp.int32),
            pltpu.VMEM((C, D), jnp.float32),
            pltpu.VMEM_SHARED((N_PAD, D), jnp.float32),
        ],
    )


def _msg_body(g_hbm, src_hbm, dst_hbm, out0_hbm, out1_hbm,
              src_b, dst_b, isems, rbufs, gsems, ssems, acc_sh):
    c = lax.axis_index("c")
    s = lax.axis_index("s")


    def pipeline(base, ch):
        def start_idx(i, q):
            pltpu.async_copy(src_hbm.at[pl.ds(base + i * CM, CM)], src_b[q],
                             isems[q])
            pltpu.async_copy(dst_hbm.at[pl.ds(base + i * CM, CM)], dst_b[q],
                             isems[q])

        def wait_idx(q):
            pltpu.make_async_copy(src_hbm.at[pl.ds(0, CM)], src_b[q],
                                  isems[q]).wait()
            pltpu.make_async_copy(dst_hbm.at[pl.ds(0, CM)], dst_b[q],
                                  isems[q]).wait()

        def start_gather(q, b):
            pltpu.async_copy(g_hbm.at[src_b[q]], rbufs[b], gsems[b])

        def wait_gather(b):
            pltpu.make_async_copy(g_hbm.at[pl.ds(0, CM)], rbufs[b],
                                  gsems[b]).wait()

        def start_scatter(q, b):
            pltpu.async_copy(rbufs[b], acc_sh.at[dst_b[q]], ssems[b],
                             add=True)

        def wait_scatter(b):
            pltpu.make_async_copy(rbufs[b], acc_sh.at[pl.ds(0, CM)],
                                  ssems[b]).wait()

        for q in range(NIDX):
            start_idx(q, q)
        for b in range(NRB):
            wait_idx(b)
            start_gather(b, b)

        def body(k, carry):
            i0 = k * NIDX
            for b in range(NIDX):
                i = i0 + b
                rb = b % NRB
                wait_gather(rb)
                start_scatter(b, rb)
                wait_idx((b + NRB) % NIDX)
                wait_scatter(rb)
                start_idx(i + NIDX, b)
                start_gather((b + NRB) % NIDX, rb)
            return carry

        n_full = (ch - 1) // NIDX
        lax.fori_loop(0, n_full, body, 0)

        last = n_full * NIDX
        wait_gather(last % NRB)
        start_scatter(last % NIDX, last % NRB)
        wait_gather((last + 1) % NRB)
        wait_scatter(last % NRB)
        for i in range(last + 2, last + NIDX):
            wait_idx(i % NIDX)

    _zero_acc(rbufs[0], acc_sh, s)
    plsc.subcore_barrier()

    @pl.when(c == 0)
    def _core0():
        pipeline(s * CH0 * CM, CH0)

    @pl.when(c == 1)
    def _core1():
        pipeline((NS * CH0 + s * CH1) * CM, CH1)

    plsc.subcore_barrier()

    @pl.when(c == 0)
    def _dump0():
        pltpu.sync_copy(acc_sh.at[pl.ds(s * SLICE, SLICE)],
                        out0_hbm.at[pl.ds(s * SLICE, SLICE)])

    @pl.when(c == 1)
    def _dump1():
        pltpu.sync_copy(acc_sh.at[pl.ds(s * SLICE, SLICE)],
                        out1_hbm.at[pl.ds(s * SLICE, SLICE)])


@functools.cache
def _msg_kernel():
    return pl.kernel(
        _msg_body,
        out_type=(jax.ShapeDtypeStruct((N_PAD, D), jnp.float32),
                  jax.ShapeDtypeStruct((N_PAD, D), jnp.float32)),
        mesh=_mesh(),
        scratch_types=[
            [pltpu.VMEM((CM,), jnp.int32)] * NIDX,
            [pltpu.VMEM((CM,), jnp.int32)] * NIDX,
            [pltpu.SemaphoreType.DMA] * NIDX,
            [pltpu.VMEM((CM, D), jnp.float32)] * NRB,
            [pltpu.SemaphoreType.DMA] * NRB,
            [pltpu.SemaphoreType.DMA] * NRB,
            pltpu.VMEM_SHARED((N_PAD, D), jnp.float32),
        ],
    )


_BLK = 1024
_GRID = N_PAD // _BLK


def _dense_first_body(x_ref, dinv_ref, wt_ref, g_ref):
    g_ref[...] = jnp.dot(x_ref[...], wt_ref[...],
                         preferred_element_type=jnp.float32) * dinv_ref[...]


def _dense_mid_body(a0_ref, a1_ref, g_ref, dinv_ref, b_ref, wt_ref, o_ref):
    t = (a0_ref[...] + a1_ref[...] + g_ref[...]) * dinv_ref[...] + b_ref[...]
    t = jnp.maximum(t, 0.0)
    o_ref[...] = jnp.dot(t, wt_ref[...],
                         preferred_element_type=jnp.float32) * dinv_ref[...]


def _pool_body(a0_ref, a1_ref, g_ref, dinv_ref, b_ref, batch_ref, out_ref,
               sums_s, counts_s):
    i = pl.program_id(0)
    h = (a0_ref[...] + a1_ref[...] + g_ref[...]) * dinv_ref[...] + b_ref[...]
    gids = lax.broadcasted_iota(jnp.int32, (G, _BLK), 0)
    onehot = (batch_ref[...].reshape(1, _BLK) == gids).astype(jnp.float32)
    part = jnp.dot(onehot, h, preferred_element_type=jnp.float32)
    cnt = jnp.broadcast_to(jnp.sum(onehot, axis=1, keepdims=True), (G, D))

    @pl.when(i == 0)
    def _init():
        sums_s[...] = part
        counts_s[...] = cnt

    @pl.when(i > 0)
    def _acc():
        sums_s[...] += part
        counts_s[...] += cnt

    @pl.when(i == _GRID - 1)
    def _fin():
        out_ref[...] = sums_s[...] / jnp.maximum(counts_s[...], 1.0)


_row_spec = pl.BlockSpec((_BLK, D), lambda i: (i, 0))
_col_spec = pl.BlockSpec((_BLK, 1), lambda i: (i, 0))
_full_spec = pl.BlockSpec((D, D), lambda i: (0, 0))
_bias_spec = pl.BlockSpec((1, D), lambda i: (0, 0))

_dense_first = pl.pallas_call(
    _dense_first_body,
    grid=(_GRID,),
    in_specs=[_row_spec, _col_spec, _full_spec],
    out_specs=_row_spec,
    out_shape=jax.ShapeDtypeStruct((N_PAD, D), jnp.float32),
)

_dense_mid = pl.pallas_call(
    _dense_mid_body,
    grid=(_GRID,),
    in_specs=[_row_spec, _row_spec, _row_spec, _col_spec, _bias_spec,
              _full_spec],
    out_specs=_row_spec,
    out_shape=jax.ShapeDtypeStruct((N_PAD, D), jnp.float32),
)

_pool = pl.pallas_call(
    _pool_body,
    grid=(_GRID,),
    in_specs=[_row_spec, _row_spec, _row_spec, _col_spec, _bias_spec,
              pl.BlockSpec((_BLK, 1), lambda i: (i, 0))],
    out_specs=pl.BlockSpec((G, D), lambda i: (0, 0)),
    out_shape=jax.ShapeDtypeStruct((G, D), jnp.float32),
    scratch_shapes=[pltpu.VMEM((G, D), jnp.float32),
                    pltpu.VMEM((G, D), jnp.float32)],
)


def _make_edges(edge_index):
    pad = N + (jnp.arange(E_IDX - E_TOT, dtype=jnp.int32) % (N_PAD - N))
    src = jnp.concatenate([edge_index[0].astype(jnp.int32), pad])
    dst = jnp.concatenate([edge_index[1].astype(jnp.int32), pad])
    return src, dst


@jax.jit
def _run_deg(edge_index):
    _, dst = _make_edges(edge_index)
    deg_parts = _deg_kernel()(dst)
    return deg_parts[:N_PAD, 0] + deg_parts[N_PAD:, 0]


@jax.jit
def _run_prep(deg, x, edge_index, W1):
    src, dst = _make_edges(edge_index)
    x_p = jnp.zeros((N_PAD, D), jnp.float32).at[:N].set(x)
    dinv = lax.rsqrt(deg + 1.0).reshape(N_PAD, 1)
    g = _dense_first(x_p, dinv, W1.T)
    return g, src, dst, dinv


@jax.jit
def _run_msg1(g, src, dst, dinv, b, Wn):
    a0, a1 = _msg_kernel()(g, src, dst)
    return _dense_mid(a0, a1, g, dinv, b.reshape(1, D), Wn.T)


_run_msg2 = _run_msg1


@jax.jit
def _run_msg3(g, src, dst, dinv, b, batch):
    a0, a1 = _msg_kernel()(g, src, dst)
    batch_p = jnp.full((N_PAD, 1), G, jnp.int32).at[:N, 0].set(
        batch.astype(jnp.int32))
    return _pool(a0, a1, g, dinv, b.reshape(1, D), batch_p)


def kernel(x, edge_index, batch, W1, b1, W2, b2, W3, b3):
    deg = _run_deg(edge_index)
    g, src, dst, dinv = _run_prep(deg, x, edge_index, W1)
    g = _run_msg1(g, src, dst, dinv, b1, W2)
    g = _run_msg2(g, src, dst, dinv, b2, W3)
    return _run_msg3(g, src, dst, dinv, b3, batch)


# speedup vs baseline: 1.7779x; 1.0034x over previous
import functools

import jax
import jax.numpy as jnp
from jax import lax
from jax.experimental import pallas as pl
from jax.experimental.pallas import tpu as pltpu
from jax.experimental.pallas import tpu_sc as plsc

N = 10000
D = 128
G = 64
N_PAD = 10240
E_TOT = 320000
NC, NS = 2, 16
NW = NC * NS
C = 128
CM = C
CH0 = 77
CH1 = 81
E_PAD = NS * (CH0 + CH1) * CM
CHUNKS_PER_W = E_PAD // (NW * C)
T_E = CHUNKS_PER_W * C
SLICE = N_PAD // NS
NIDX = 4
NRB = 2
E_IDX = E_PAD + (NIDX - 1) * CM

@functools.cache
def _mesh():
    return plsc.VectorSubcoreMesh(core_axis_name="c", subcore_axis_name="s",
                                  num_cores=NC, num_subcores=NS)


def _fill(buf, rows, value):
    def fbody(j, carry):
        for f in range(D // 16):
            buf[j, pl.ds(16 * f, 16)] = jnp.full((16,), value, jnp.float32)
        return carry

    lax.fori_loop(0, rows, fbody, 0)


def _zero_acc(buf, acc_sh, s):
    _fill(buf, CM, 0.0)
    for t in range(SLICE // CM):
        pltpu.sync_copy(buf, acc_sh.at[pl.ds(s * SLICE + t * CM, CM)])


def _deg_body(dst_hbm, out_hbm, dst_v, ones_v, acc_sh):
    c = lax.axis_index("c")
    s = lax.axis_index("s")
    wid = s * NC + c
    _zero_acc(ones_v, acc_sh, s)
    _fill(ones_v, C, 1.0)
    plsc.subcore_barrier()

    def body(i, carry):
        base = wid * T_E + i * C
        pltpu.sync_copy(dst_hbm.at[pl.ds(base, C)], dst_v)
        pltpu.sync_copy(ones_v, acc_sh.at[dst_v], add=True)
        return carry

    lax.fori_loop(0, CHUNKS_PER_W, body, 0)
    plsc.subcore_barrier()
    pltpu.sync_copy(acc_sh.at[pl.ds(s * SLICE, SLICE)],
                    out_hbm.at[pl.ds(c * N_PAD + s * SLICE, SLICE)])


@functools.cache
def _deg_kernel():
    return pl.kernel(
        _deg_body,
        out_type=jax.ShapeDtypeStruct((NC * N_PAD, D), jnp.float32),
        mesh=_mesh(),
        scratch_types=[
            pltpu.VMEM((C,), jnp.int32),
            pltpu.VMEM((C, D), jnp.float32),
            pltpu.VMEM_SHARED((N_PAD, D), jnp.float32),
        ],
    )


def _msg_body(g_hbm, src_hbm, dst_hbm, out0_hbm, out1_hbm,
              src_b, dst_b, isems, rbufs, gsems, ssems, acc_sh):
    c = lax.axis_index("c")
    s = lax.axis_index("s")


    def pipeline(base, ch):
        def start_idx(i, q):
            pltpu.async_copy(src_hbm.at[pl.ds(base + i * CM, CM)], src_b[q],
                             isems[q])
            pltpu.async_copy(dst_hbm.at[pl.ds(base + i * CM, CM)], dst_b[q],
                             isems[q])

        def wait_idx(q):
            pltpu.make_async_copy(src_hbm.at[pl.ds(0, CM)], src_b[q],
                                  isems[q]).wait()
            pltpu.make_async_copy(dst_hbm.at[pl.ds(0, CM)], dst_b[q],
                                  isems[q]).wait()

        def start_gather(q, b):
            pltpu.async_copy(g_hbm.at[src_b[q]], rbufs[b], gsems[b])

        def wait_gather(b):
            pltpu.make_async_copy(g_hbm.at[pl.ds(0, CM)], rbufs[b],
                                  gsems[b]).wait()

        def start_scatter(q, b):
            pltpu.async_copy(rbufs[b], acc_sh.at[dst_b[q]], ssems[b],
                             add=True)

        def wait_scatter(b):
            pltpu.make_async_copy(rbufs[b], acc_sh.at[pl.ds(0, CM)],
                                  ssems[b]).wait()

        for q in range(NIDX):
            start_idx(q, q)
        for b in range(NRB):
            wait_idx(b)
            start_gather(b, b)

        def body(k, carry):
            i0 = k * NIDX
            for b in range(NIDX):
                i = i0 + b
                rb = b % NRB
                wait_gather(rb)
                start_scatter(b, rb)
                wait_idx((b + NRB) % NIDX)
                wait_scatter(rb)
                start_idx(i + NIDX, b)
                start_gather((b + NRB) % NIDX, rb)
            return carry

        n_full = (ch - 1) // NIDX
        lax.fori_loop(0, n_full, body, 0)

        last = n_full * NIDX
        wait_gather(last % NRB)
        start_scatter(last % NIDX, last % NRB)
        wait_gather((last + 1) % NRB)
        wait_scatter(last % NRB)
        for i in range(last + 2, last + NIDX):
            wait_idx(i % NIDX)

    _zero_acc(rbufs[0], acc_sh, s)
    plsc.subcore_barrier()

    @pl.when(c == 0)
    def _core0():
        pipeline(s * CH0 * CM, CH0)

    @pl.when(c == 1)
    def _core1():
        pipeline((NS * CH0 + s * CH1) * CM, CH1)

    plsc.subcore_barrier()

    @pl.when(c == 0)
    def _dump0():
        pltpu.sync_copy(acc_sh.at[pl.ds(s * SLICE, SLICE)],
                        out0_hbm.at[pl.ds(s * SLICE, SLICE)])

    @pl.when(c == 1)
    def _dump1():
        pltpu.sync_copy(acc_sh.at[pl.ds(s * SLICE, SLICE)],
                        out1_hbm.at[pl.ds(s * SLICE, SLICE)])


@functools.cache
def _msg_kernel():
    return pl.kernel(
        _msg_body,
        out_type=(jax.ShapeDtypeStruct((N_PAD, D), jnp.float32),
                  jax.ShapeDtypeStruct((N_PAD, D), jnp.float32)),
        mesh=_mesh(),
        scratch_types=[
            [pltpu.VMEM((CM,), jnp.int32)] * NIDX,
            [pltpu.VMEM((CM,), jnp.int32)] * NIDX,
            [pltpu.SemaphoreType.DMA] * NIDX,
            [pltpu.VMEM((CM, D), jnp.float32)] * NRB,
            [pltpu.SemaphoreType.DMA] * NRB,
            [pltpu.SemaphoreType.DMA] * NRB,
            pltpu.VMEM_SHARED((N_PAD, D), jnp.float32),
        ],
    )


_BLK = 1024
_GRID = N_PAD // _BLK


def _dense_first_body(x_ref, dinv_ref, wt_ref, g_ref):
    g_ref[...] = jnp.dot(x_ref[...], wt_ref[...],
                         preferred_element_type=jnp.float32) * dinv_ref[...]


def _dense_mid_body(a0_ref, a1_ref, g_ref, dinv_ref, b_ref, wt_ref, o_ref):
    t = (a0_ref[...] + a1_ref[...] + g_ref[...]) * dinv_ref[...] + b_ref[...]
    t = jnp.maximum(t, 0.0)
    o_ref[...] = jnp.dot(t, wt_ref[...],
                         preferred_element_type=jnp.float32) * dinv_ref[...]


def _pool_body(a0_ref, a1_ref, g_ref, dinv_ref, b_ref, batch_ref, out_ref,
               sums_s, counts_s):
    i = pl.program_id(0)
    h = (a0_ref[...] + a1_ref[...] + g_ref[...]) * dinv_ref[...] + b_ref[...]
    gids = lax.broadcasted_iota(jnp.int32, (G, _BLK), 0)
    onehot = (batch_ref[...].reshape(1, _BLK) == gids).astype(jnp.float32)
    part = jnp.dot(onehot, h, preferred_element_type=jnp.float32)
    cnt = jnp.broadcast_to(jnp.sum(onehot, axis=1, keepdims=True), (G, D))

    @pl.when(i == 0)
    def _init():
        sums_s[...] = part
        counts_s[...] = cnt

    @pl.when(i > 0)
    def _acc():
        sums_s[...] += part
        counts_s[...] += cnt

    @pl.when(i == _GRID - 1)
    def _fin():
        out_ref[...] = sums_s[...] / jnp.maximum(counts_s[...], 1.0)


_row_spec = pl.BlockSpec((_BLK, D), lambda i: (i, 0))
_col_spec = pl.BlockSpec((_BLK, 1), lambda i: (i, 0))
_full_spec = pl.BlockSpec((D, D), lambda i: (0, 0))
_bias_spec = pl.BlockSpec((1, D), lambda i: (0, 0))

_dense_first = pl.pallas_call(
    _dense_first_body,
    grid=(_GRID,),
    in_specs=[_row_spec, _col_spec, _full_spec],
    out_specs=_row_spec,
    out_shape=jax.ShapeDtypeStruct((N_PAD, D), jnp.float32),
)

_dense_mid = pl.pallas_call(
    _dense_mid_body,
    grid=(_GRID,),
    in_specs=[_row_spec, _row_spec, _row_spec, _col_spec, _bias_spec,
              _full_spec],
    out_specs=_row_spec,
    out_shape=jax.ShapeDtypeStruct((N_PAD, D), jnp.float32),
)

_pool = pl.pallas_call(
    _pool_body,
    grid=(_GRID,),
    in_specs=[_row_spec, _row_spec, _row_spec, _col_spec, _bias_spec,
              pl.BlockSpec((_BLK, 1), lambda i: (i, 0))],
    out_specs=pl.BlockSpec((G, D), lambda i: (0, 0)),
    out_shape=jax.ShapeDtypeStruct((G, D), jnp.float32),
    scratch_shapes=[pltpu.VMEM((G, D), jnp.float32),
                    pltpu.VMEM((G, D), jnp.float32)],
)


def _make_edges(edge_index):
    pad = N + (jnp.arange(E_IDX - E_TOT, dtype=jnp.int32) % (N_PAD - N))
    src = jnp.concatenate([edge_index[0].astype(jnp.int32), pad])
    dst = jnp.concatenate([edge_index[1].astype(jnp.int32), pad])
    return src, dst


@jax.jit
def _run_deg(edge_index):
    _, dst = _make_edges(edge_index)
    deg_parts = _deg_kernel()(dst)
    return deg_parts[:N_PAD, 0] + deg_parts[N_PAD:, 0]


@jax.jit
def _run_prep(deg, x, edge_index, W1):
    src, dst = _make_edges(edge_index)
    x_p = jnp.zeros((N_PAD, D), jnp.float32).at[:N].set(x)
    dinv = lax.rsqrt(deg + 1.0).reshape(N_PAD, 1)
    g = _dense_first(x_p, dinv, W1.T)
    return g, src, dst, dinv


@jax.jit
def _run_msg1(g, src, dst, dinv, b, Wn):
    a0, a1 = _msg_kernel()(g, src, dst)
    return _dense_mid(a0, a1, g, dinv, b.reshape(1, D), Wn.T)


_run_msg2 = _run_msg1


@jax.jit
def _run_msg3(g, src, dst, dinv, b, batch):
    a0, a1 = _msg_kernel()(g, src, dst)
    batch_p = jnp.full((N_PAD, 1), G, jnp.int32).at[:N, 0].set(
        batch.astype(jnp.int32))
    return _pool(a0, a1, g, dinv, b.reshape(1, D), batch_p)


def kernel(x, edge_index, batch, W1, b1, W2, b2, W3, b3):
    deg = _run_deg(edge_index)
    g, src, dst, dinv = _run_prep(deg, x, edge_index, W1)
    g = _run_msg1(g, src, dst, dinv, b1, W2)
    g = _run_msg2(g, src, dst, dinv, b2, W3)
    return _run_msg3(g, src, dst, dinv, b3, batch)
